# baseline (speedup 1.0000x reference)
import jax
import jax.numpy as jnp
from jax import lax
from jax.experimental import pallas as pl
from jax.experimental.pallas import tpu as pltpu
from jax.experimental.pallas import tpu_sc as plsc

MAXLEN = 2048
EMBED_DIM = 128
BATCH = 32

NUM_CORES = 2
NUM_SUBCORES = 16
LANES = 16
NW = NUM_CORES * NUM_SUBCORES
P = MAXLEN // NW


def _emb_body(x_hbm, tok_hbm, pos_hbm, out_hbm, idx_v, rows_v, pos_v, sem):
    wid = lax.axis_index("s") * NUM_CORES + lax.axis_index("c")
    pbase = wid * P
    pltpu.sync_copy(pos_hbm.at[pl.ds(pbase, P)], pos_v)

    def batch_body(b, carry):
        row0 = b * MAXLEN + pbase
        pltpu.sync_copy(x_hbm.at[pl.ds(row0, P)], idx_v)
        pltpu.async_copy(tok_hbm.at[idx_v], rows_v, sem).wait()

        def add_body(r, c2):
            for j in range(EMBED_DIM // LANES):
                s = pl.ds(j * LANES, LANES)
                rows_v[r, s] = rows_v[r, s] + pos_v[r, s]
            return c2

        lax.fori_loop(0, P, add_body, 0)
        pltpu.sync_copy(rows_v, out_hbm.at[pl.ds(row0, P)])
        return carry

    lax.fori_loop(0, BATCH, batch_body, 0)


def kernel(x, token_table, pos_table):
    x_flat = x.reshape(-1).astype(jnp.int32)
    mesh = plsc.VectorSubcoreMesh(core_axis_name="c", subcore_axis_name="s")
    f = pl.kernel(
        _emb_body,
        mesh=mesh,
        out_type=jax.ShapeDtypeStruct((BATCH * MAXLEN, EMBED_DIM), jnp.float32),
        scratch_types=[
            pltpu.VMEM((P,), jnp.int32),
            pltpu.VMEM((P, EMBED_DIM), jnp.float32),
            pltpu.VMEM((P, EMBED_DIM), jnp.float32),
            pltpu.SemaphoreType.DMA,
        ],
    )
    out = f(x_flat, token_table, pos_table)
    return out.reshape(BATCH, MAXLEN, EMBED_DIM)


# speedup vs baseline: 1.9862x; 1.9862x over previous
import jax
import jax.numpy as jnp
from jax import lax
from jax.experimental import pallas as pl
from jax.experimental.pallas import tpu as pltpu
from jax.experimental.pallas import tpu_sc as plsc

MAXLEN = 2048
EMBED_DIM = 128
BATCH = 32

NUM_CORES = 2
NUM_SUBCORES = 16
LANES = 16
NW = NUM_CORES * NUM_SUBCORES
P = MAXLEN // NW
NBUF = 8
LOOKAHEAD = 4


def _emb_body(x_hbm, tok_hbm, pos_hbm, out_hbm,
              idx_all, rows_v, pos_v, gsem, ssem, psem, isem):
    wid = lax.axis_index("s") * NUM_CORES + lax.axis_index("c")
    pbase = wid * P

    pos_cp = pltpu.async_copy(pos_hbm.at[pl.ds(pbase, P)], pos_v, psem)
    idx_cps = [
        pltpu.async_copy(
            x_hbm.at[pl.ds(b * MAXLEN + pbase, P)], idx_all.at[b], isem)
        for b in range(BATCH)
    ]
    for cp in idx_cps:
        cp.wait()

    def gather(b):
        return pltpu.make_async_copy(
            tok_hbm.at[idx_all.at[b]], rows_v.at[b % NBUF], gsem.at[b % NBUF])

    def store(b):
        return pltpu.make_async_copy(
            rows_v.at[b % NBUF],
            out_hbm.at[pl.ds(b * MAXLEN + pbase, P)],
            ssem.at[b % NBUF])

    for b in range(LOOKAHEAD):
        gather(b).start()
    pos_cp.wait()

    for b in range(BATCH):
        s = b % NBUF
        gather(b).wait()

        def add_body(r, carry):
            for j in range(EMBED_DIM // LANES):
                sl = pl.ds(j * LANES, LANES)
                rows_v[s, r, sl] = rows_v[s, r, sl] + pos_v[r, sl]
            return carry

        lax.fori_loop(0, P, add_body, 0)
        store(b).start()
        if b + LOOKAHEAD < BATCH:
            nb = b + LOOKAHEAD
            if nb >= NBUF:
                store(nb - NBUF).wait()
            gather(nb).start()

    for b in range(BATCH - NBUF, BATCH):
        store(b).wait()


def kernel(x, token_table, pos_table):
    x32 = x.reshape(-1).astype(jnp.int32)
    mesh = plsc.VectorSubcoreMesh(core_axis_name="c", subcore_axis_name="s")
    f = pl.kernel(
        _emb_body,
        mesh=mesh,
        out_type=jax.ShapeDtypeStruct((BATCH * MAXLEN, EMBED_DIM), jnp.float32),
        scratch_types=[
            pltpu.VMEM((BATCH, P), jnp.int32),
            pltpu.VMEM((NBUF, P, EMBED_DIM), jnp.float32),
            pltpu.VMEM((P, EMBED_DIM), jnp.float32),
            pltpu.SemaphoreType.DMA((NBUF,)),
            pltpu.SemaphoreType.DMA((NBUF,)),
            pltpu.SemaphoreType.DMA,
            pltpu.SemaphoreType.DMA,
        ],
    )
    out = f(x32, token_table, pos_table)
    return out.reshape(BATCH, MAXLEN, EMBED_DIM)
